# speedup vs baseline: 4.1791x; 4.1791x over previous
import functools

import jax
import jax.numpy as jnp
from jax import lax
from jax.experimental import pallas as pl
from jax.experimental.pallas import tpu as pltpu
from jax.experimental.pallas import tpu_sc as plsc

B, S, D = 4, 8192, 1024
W = 4096
NC, NS, L = 2, 16, 16
NW = NC * NS
ROWS_PER_W = (B * W) // NW
CHUNK = 32
NCHUNK = ROWS_PER_W // CHUNK

_mesh = plsc.VectorSubcoreMesh(core_axis_name="c", subcore_axis_name="s")


@functools.partial(
    pl.kernel,
    out_type=jax.ShapeDtypeStruct((B * W, D), jnp.float32),
    mesh=_mesh,
    scratch_types=[
        pltpu.VMEM((ROWS_PER_W,), jnp.int32),
        pltpu.VMEM((2, CHUNK, D), jnp.float32),
        pltpu.SemaphoreType.DMA,
        pltpu.SemaphoreType.DMA,
        pltpu.SemaphoreType.DMA,
        pltpu.SemaphoreType.DMA,
    ],
)
def _gather_rows(table_hbm, idx_hbm, out_hbm, idx_v, rows_v, g0, g1, o0, o1):
    wid = lax.axis_index("s") * NC + lax.axis_index("c")
    base = wid * ROWS_PER_W
    gsem = [g0, g1]
    osem = [o0, o1]

    pltpu.sync_copy(idx_hbm.at[pl.ds(base, ROWS_PER_W)], idx_v)
    off = (base // W) * S + 1
    for i in range(ROWS_PER_W // L):
        sl = pl.ds(i * L, L)
        idx_v[sl] = idx_v[sl] + off

    def start_gather(c, buf):
        return pltpu.async_copy(
            table_hbm.at[idx_v.at[pl.ds(c * CHUNK, CHUNK)]],
            rows_v.at[buf],
            gsem[buf],
        )

    def start_write(c, buf):
        return pltpu.async_copy(
            rows_v.at[buf],
            out_hbm.at[pl.ds(base + c * CHUNK, CHUNK)],
            osem[buf],
        )

    gcp = [None, None]
    ocp = [None, None]
    gcp[0] = start_gather(0, 0)
    for c in range(NCHUNK):
        buf = c % 2
        nbuf = (c + 1) % 2
        if c + 1 < NCHUNK:
            if ocp[nbuf] is not None:
                ocp[nbuf].wait()
            gcp[nbuf] = start_gather(c + 1, nbuf)
        gcp[buf].wait()
        ocp[buf] = start_write(c, buf)
    ocp[0].wait()
    ocp[1].wait()


def kernel(outputs, word_index, word_attention_mask):
    table = outputs.reshape(B * S, D)
    idx = word_index.astype(jnp.int32).reshape(B * W)
    gathered = _gather_rows(table, idx)
    return gathered.reshape(B, W, D), word_attention_mask
